# Initial kernel scaffold; baseline (speedup 1.0000x reference)
import jax
import jax.numpy as jnp
from jax.experimental import pallas as pl


def kernel(X, batch_ids, static_graph_features, W1, b1, W2, b2, Wout, bout):
    raise NotImplementedError("write your pallas kernel here")


# speedup vs baseline: 3.0125x; 3.0125x over previous
import functools

import jax
import jax.numpy as jnp
from jax import lax
from jax.experimental import pallas as pl
from jax.experimental.pallas import tpu as pltpu
from jax.experimental.pallas import tpu_sc as plsc

N, H, S, G, O = 100000, 256, 64, 512, 128
D = H + S

NC, NS = 2, 16
NW = NC * NS
CHUNK = 125
NCHUNK = N // CHUNK
CPW = NCHUNK // NW
IPAD = 128
TRASH = G
ACC_ROWS = 544
ZROWS = ACC_ROWS // NS
CNT_W = 16


def _sc_pool(x, ids_pad, zsum, zcnt, ones):
    mesh = plsc.VectorSubcoreMesh(core_axis_name="c", subcore_axis_name="s")

    @functools.partial(
        pl.kernel,
        out_type=[
            jax.ShapeDtypeStruct((NC, G, H), jnp.float32),
            jax.ShapeDtypeStruct((NC, G, CNT_W), jnp.float32),
        ],
        mesh=mesh,
        scratch_types=[
            pltpu.VMEM((IPAD, H), jnp.float32),
            pltpu.VMEM((IPAD,), jnp.int32),
            pltpu.VMEM((IPAD, CNT_W), jnp.float32),
            pltpu.VMEM_SHARED((ACC_ROWS, H), jnp.float32),
            pltpu.VMEM_SHARED((ACC_ROWS, CNT_W), jnp.float32),
        ],
        compiler_params=pltpu.CompilerParams(use_tc_tiling_on_sc=False),
    )
    def pool(x_hbm, ids_hbm, zsum_hbm, zcnt_hbm, ones_hbm,
             sums_out, cnts_out, rows_v, ids_v, ones_v, acc_sh, cnt_sh):
        c = lax.axis_index("c")
        s = lax.axis_index("s")
        wid = s * NC + c

        pltpu.sync_copy(zsum_hbm.at[pl.ds(s * ZROWS, ZROWS)],
                        acc_sh.at[pl.ds(s * ZROWS, ZROWS)])
        pltpu.sync_copy(zcnt_hbm.at[pl.ds(s * ZROWS, ZROWS)],
                        cnt_sh.at[pl.ds(s * ZROWS, ZROWS)])
        pltpu.sync_copy(ones_hbm, ones_v)
        pltpu.sync_copy(zsum_hbm.at[pl.ds(0, IPAD - CHUNK)],
                        rows_v.at[pl.ds(CHUNK, IPAD - CHUNK)])
        plsc.subcore_barrier()

        def body(t, carry):
            gc = wid * CPW + t
            pltpu.sync_copy(ids_hbm.at[gc], ids_v)
            pltpu.sync_copy(x_hbm.at[pl.ds(gc * CHUNK, CHUNK)],
                            rows_v.at[pl.ds(0, CHUNK)])
            pltpu.sync_copy(rows_v, acc_sh.at[ids_v], add=True)
            pltpu.sync_copy(ones_v, cnt_sh.at[ids_v], add=True)
            return carry

        lax.fori_loop(0, CPW, body, 0)
        plsc.subcore_barrier()

        @pl.when(s == 0)
        def _():
            pltpu.sync_copy(acc_sh.at[pl.ds(0, G)], sums_out.at[c])
            pltpu.sync_copy(cnt_sh.at[pl.ds(0, G)], cnts_out.at[c])

    return pool(x, ids_pad, zsum, zcnt, ones)


def _elu(v):
    return jnp.where(v > 0.0, v, jnp.exp(jnp.minimum(v, 0.0)) - 1.0)


def _dot(a, b):
    return jnp.dot(a, b, preferred_element_type=jnp.float32,
                   precision=lax.Precision.HIGHEST)


def _mlp_body(sums_ref, cnts_ref, st_ref, w1_ref, b1_ref, w2_ref, b2_ref,
              wo_ref, bo_ref, out_ref):
    sums = sums_ref[0] + sums_ref[1]
    cnt = cnts_ref[0, :, 0:1] + cnts_ref[1, :, 0:1]
    pooled = sums / jnp.maximum(cnt, 1.0)
    h = (_dot(pooled, w1_ref[0:H, :]) + _dot(st_ref[...], w1_ref[H:D, :])
         + b1_ref[...])
    h = _elu(h)
    h = _elu(_dot(h, w2_ref[...]) + b2_ref[...])
    out_ref[...] = _dot(h, wo_ref[...]) + bo_ref[...]


def kernel(X, batch_ids, static_graph_features, W1, b1, W2, b2, Wout, bout):
    ids = batch_ids.astype(jnp.int32).reshape(NCHUNK, CHUNK)
    ids_pad = jnp.full((NCHUNK, IPAD), TRASH, jnp.int32).at[:, :CHUNK].set(ids)
    zsum = jnp.zeros((ACC_ROWS, H), jnp.float32)
    zcnt = jnp.zeros((ACC_ROWS, CNT_W), jnp.float32)
    ones = jnp.ones((IPAD, CNT_W), jnp.float32)
    sums2, cnts2 = _sc_pool(X, ids_pad, zsum, zcnt, ones)
    return pl.pallas_call(
        _mlp_body,
        out_shape=jax.ShapeDtypeStruct((G, O), jnp.float32),
    )(sums2, cnts2, static_graph_features, W1, b1, W2, b2, Wout, bout)


# speedup vs baseline: 3.5214x; 1.1689x over previous
import functools

import jax
import jax.numpy as jnp
from jax import lax
from jax.experimental import pallas as pl
from jax.experimental.pallas import tpu as pltpu
from jax.experimental.pallas import tpu_sc as plsc

N, H, S, G, O = 100000, 256, 64, 512, 128
D = H + S

NC, NS = 2, 16
NW = NC * NS
CHUNK = 125
NCHUNK = N // CHUNK
CPW = NCHUNK // NW
IPAD = 128
TRASH = G
ACC_ROWS = 544
ZROWS = ACC_ROWS // NS
CNT_W = 16


def _sc_pool(x, ids_pad, zsum, zcnt, ones):
    mesh = plsc.VectorSubcoreMesh(core_axis_name="c", subcore_axis_name="s")

    @functools.partial(
        pl.kernel,
        out_type=[
            jax.ShapeDtypeStruct((NC, G, H), jnp.float32),
            jax.ShapeDtypeStruct((NC, G, CNT_W), jnp.float32),
        ],
        mesh=mesh,
        scratch_types=[
            pltpu.VMEM((IPAD, H), jnp.float32),
            pltpu.VMEM((IPAD, H), jnp.float32),
            pltpu.VMEM((IPAD,), jnp.int32),
            pltpu.VMEM((IPAD,), jnp.int32),
            pltpu.VMEM((IPAD, CNT_W), jnp.float32),
            pltpu.VMEM_SHARED((ACC_ROWS, H), jnp.float32),
            pltpu.VMEM_SHARED((ACC_ROWS, CNT_W), jnp.float32),
            pltpu.SemaphoreType.DMA,
            pltpu.SemaphoreType.DMA,
        ],
        compiler_params=pltpu.CompilerParams(use_tc_tiling_on_sc=False),
    )
    def pool(x_hbm, ids_hbm, zsum_hbm, zcnt_hbm, ones_hbm,
             sums_out, cnts_out, rows0, rows1, ids0, ids1, ones_v,
             acc_sh, cnt_sh, sem0, sem1):
        c = lax.axis_index("c")
        s = lax.axis_index("s")
        wid = s * NC + c
        base = wid * CPW
        rows_b, ids_b, sems = (rows0, rows1), (ids0, ids1), (sem0, sem1)

        def start_load(t, b):
            gc = base + t
            pltpu.async_copy(ids_hbm.at[gc], ids_b[b], sems[b])
            pltpu.async_copy(x_hbm.at[pl.ds(gc * CHUNK, CHUNK)],
                             rows_b[b].at[pl.ds(0, CHUNK)], sems[b])

        def wait_load(b):
            pltpu.make_async_copy(ids_hbm.at[0], ids_b[b], sems[b]).wait()
            pltpu.make_async_copy(x_hbm.at[pl.ds(0, CHUNK)],
                                  rows_b[b].at[pl.ds(0, CHUNK)],
                                  sems[b]).wait()

        start_load(0, 0)
        start_load(1, 1)
        pltpu.sync_copy(zsum_hbm.at[pl.ds(s * ZROWS, ZROWS)],
                        acc_sh.at[pl.ds(s * ZROWS, ZROWS)])
        pltpu.sync_copy(zcnt_hbm.at[pl.ds(s * ZROWS, ZROWS)],
                        cnt_sh.at[pl.ds(s * ZROWS, ZROWS)])
        pltpu.sync_copy(ones_hbm, ones_v)
        pltpu.sync_copy(zsum_hbm.at[pl.ds(0, IPAD - CHUNK)],
                        rows0.at[pl.ds(CHUNK, IPAD - CHUNK)])
        pltpu.sync_copy(zsum_hbm.at[pl.ds(0, IPAD - CHUNK)],
                        rows1.at[pl.ds(CHUNK, IPAD - CHUNK)])
        plsc.subcore_barrier()

        def body(tt, carry):
            for b in range(2):
                t = 2 * tt + b

                @pl.when(t < CPW)
                def _process():
                    wait_load(b)
                    pltpu.sync_copy(rows_b[b], acc_sh.at[ids_b[b]], add=True)
                    pltpu.sync_copy(ones_v, cnt_sh.at[ids_b[b]], add=True)

                    @pl.when(t + 2 < CPW)
                    def _prefetch():
                        start_load(t + 2, b)

            return carry

        lax.fori_loop(0, (CPW + 1) // 2, body, 0)
        plsc.subcore_barrier()

        @pl.when(s == 0)
        def _():
            pltpu.sync_copy(acc_sh.at[pl.ds(0, G)], sums_out.at[c])
            pltpu.sync_copy(cnt_sh.at[pl.ds(0, G)], cnts_out.at[c])

    return pool(x, ids_pad, zsum, zcnt, ones)


def _elu(v):
    return jnp.where(v > 0.0, v, jnp.exp(jnp.minimum(v, 0.0)) - 1.0)


def _dot(a, b):
    return jnp.dot(a, b, preferred_element_type=jnp.float32,
                   precision=lax.Precision.HIGHEST)


def _mlp_body(sums_ref, cnts_ref, st_ref, w1_ref, b1_ref, w2_ref, b2_ref,
              wo_ref, bo_ref, out_ref):
    sums = sums_ref[0] + sums_ref[1]
    cnt = cnts_ref[0, :, 0:1] + cnts_ref[1, :, 0:1]
    pooled = sums / jnp.maximum(cnt, 1.0)
    h = (_dot(pooled, w1_ref[0:H, :]) + _dot(st_ref[...], w1_ref[H:D, :])
         + b1_ref[...])
    h = _elu(h)
    h = _elu(_dot(h, w2_ref[...]) + b2_ref[...])
    out_ref[...] = _dot(h, wo_ref[...]) + bo_ref[...]


def kernel(X, batch_ids, static_graph_features, W1, b1, W2, b2, Wout, bout):
    ids = batch_ids.astype(jnp.int32).reshape(NCHUNK, CHUNK)
    ids_pad = jnp.full((NCHUNK, IPAD), TRASH, jnp.int32).at[:, :CHUNK].set(ids)
    zsum = jnp.zeros((ACC_ROWS, H), jnp.float32)
    zcnt = jnp.zeros((ACC_ROWS, CNT_W), jnp.float32)
    ones = jnp.ones((IPAD, CNT_W), jnp.float32)
    sums2, cnts2 = _sc_pool(X, ids_pad, zsum, zcnt, ones)
    return pl.pallas_call(
        _mlp_body,
        out_shape=jax.ShapeDtypeStruct((G, O), jnp.float32),
    )(sums2, cnts2, static_graph_features, W1, b1, W2, b2, Wout, bout)
